# baseline (speedup 1.0000x reference)
import functools

import jax
import jax.numpy as jnp
from jax import lax
from jax.experimental import pallas as pl
from jax.experimental.pallas import tpu as pltpu
from jax.experimental.pallas import tpu_sc as plsc

_B, _L = 8, 2048
_LANES = 16
_CHUNKS = _L // _LANES
_EXT_MAX_ID = 20030

_M0 = _LANES
_T0 = _LANES + _L


def _sc_body(mt_hbm, out_hbm, buf_v, out_v):
    wid = lax.axis_index("s")

    @pl.when(wid < _B)
    def _():
        buf_v[pl.ds(0, _LANES)] = jnp.zeros((_LANES,), jnp.int32)
        pltpu.sync_copy(mt_hbm.at[pl.ds(wid * 2 * _L, 2 * _L)],
                        buf_v.at[pl.ds(_LANES, 2 * _L)])

        lane15 = jnp.full((_LANES,), 15, jnp.int32)

        def local_cummax(base):
            idx = lax.iota(jnp.int32, _LANES) + base
            m_cur = buf_v[pl.ds(base + _M0, _LANES)]
            m_prev = buf_v[pl.ds(base + _M0 - 1, _LANES)]
            t_prev = buf_v[pl.ds(base + _T0 - 1, _LANES)]
            a = (m_cur == 1) & (m_prev == 1) & (t_prev <= _EXT_MAX_ID)
            return plsc.cummax(jnp.where(a, -1, idx))

        def body(i, carry):
            lz_v, n_v, yloc = carry
            base = i * _LANES
            idx = lax.iota(jnp.int32, _LANES) + base
            y = jnp.maximum(yloc, lz_v)
            par = (idx - y) & 1
            n_cnt = plsc.all_reduce_population_count(par == 0)
            out_v[pl.ds(base, _LANES)] = plsc.cumsum(1 - par) + n_v - 1
            return (y.at[lane15].get(mode="promise_in_bounds"),
                    n_v + n_cnt, local_cummax(base + _LANES))

        lax.fori_loop(0, _CHUNKS, body,
                      (jnp.full((_LANES,), -1, jnp.int32),
                       jnp.zeros((_LANES,), jnp.int32),
                       local_cummax(0)), unroll=1)
        pltpu.sync_copy(out_v, out_hbm.at[pl.ds(wid * _L, _L)])


_sc_chunker = functools.partial(
    pl.kernel,
    out_type=jax.ShapeDtypeStruct((_B * _L,), jnp.int32),
    mesh=plsc.VectorSubcoreMesh(core_axis_name="c", subcore_axis_name="s",
                                num_cores=1, num_subcores=16),
    compiler_params=pltpu.CompilerParams(needs_layout_passes=False),
    scratch_types=[
        pltpu.VMEM((2 * _L + 2 * _LANES,), jnp.int32),
        pltpu.VMEM((_L,), jnp.int32),
    ],
)(_sc_body)


def kernel(inp, regular_tokens_mask, token_ids):
    del inp
    mt = jnp.stack([regular_tokens_mask, token_ids],
                   axis=1).reshape(2 * _B * _L)
    return _sc_chunker(mt).reshape(_B, _L)


# speedup vs baseline: 1.0097x; 1.0097x over previous
import functools

import jax
import jax.numpy as jnp
from jax import lax
from jax.experimental import pallas as pl
from jax.experimental.pallas import tpu as pltpu
from jax.experimental.pallas import tpu_sc as plsc

_B, _L = 8, 2048
_LANES = 16
_CHUNKS = _L // _LANES
_EXT_MAX_ID = 20030

_M0 = _LANES
_T0 = _LANES + _L


def _sc_body(mt_hbm, out_hbm, buf_v, out_v):
    wid = lax.axis_index("s")

    @pl.when(wid < _B)
    def _():
        buf_v[pl.ds(0, _LANES)] = jnp.zeros((_LANES,), jnp.int32)
        pltpu.sync_copy(mt_hbm.at[pl.ds(wid * 2 * _L, 2 * _L)],
                        buf_v.at[pl.ds(_LANES, 2 * _L)])

        lane15 = jnp.full((_LANES,), 15, jnp.int32)

        def body(i, carry):
            lz_v, n_v = carry
            base = i * _LANES
            idx = lax.iota(jnp.int32, _LANES) + base
            m_cur = buf_v[pl.ds(base + _M0, _LANES)]
            m_prev = buf_v[pl.ds(base + _M0 - 1, _LANES)]
            t_prev = buf_v[pl.ds(base + _T0 - 1, _LANES)]
            a = (m_cur == 1) & (m_prev == 1) & (t_prev <= _EXT_MAX_ID)
            v = jnp.where(a, -1, idx)
            y = jnp.maximum(plsc.cummax(v), lz_v)
            n = 1 - ((idx - y) & 1)
            c = plsc.cumsum(n) + n_v
            out_v[pl.ds(base, _LANES)] = c - 1
            return (y.at[lane15].get(mode="promise_in_bounds"),
                    c.at[lane15].get(mode="promise_in_bounds"))

        lax.fori_loop(0, _CHUNKS, body,
                      (jnp.full((_LANES,), -1, jnp.int32),
                       jnp.zeros((_LANES,), jnp.int32)), unroll=1)
        pltpu.sync_copy(out_v, out_hbm.at[pl.ds(wid * _L, _L)])


_sc_chunker = functools.partial(
    pl.kernel,
    out_type=jax.ShapeDtypeStruct((_B * _L,), jnp.int32),
    mesh=plsc.VectorSubcoreMesh(core_axis_name="c", subcore_axis_name="s",
                                num_cores=1, num_subcores=16),
    compiler_params=pltpu.CompilerParams(needs_layout_passes=False),
    scratch_types=[
        pltpu.VMEM((2 * _L + 2 * _LANES,), jnp.int32),
        pltpu.VMEM((_L,), jnp.int32),
    ],
)(_sc_body)


def kernel(inp, regular_tokens_mask, token_ids):
    del inp
    mt = jnp.stack([regular_tokens_mask, token_ids],
                   axis=1).reshape(2 * _B * _L)
    return _sc_chunker(mt).reshape(_B, _L)


# speedup vs baseline: 1.0120x; 1.0023x over previous
import functools

import jax
import jax.numpy as jnp
from jax import lax
from jax.experimental import pallas as pl
from jax.experimental.pallas import tpu as pltpu
from jax.experimental.pallas import tpu_sc as plsc

_B, _L = 8, 2048
_LANES = 16
_CHUNKS = _L // _LANES
_EXT_MAX_ID = 20030

_M0 = _LANES
_T0 = _LANES + _L


def _sc_body(mt_hbm, out_hbm, buf_v, out_v):
    wid = lax.axis_index("s")

    @pl.when(wid < _B)
    def _():
        buf_v[pl.ds(0, _LANES)] = jnp.zeros((_LANES,), jnp.int32)
        pltpu.sync_copy(mt_hbm.at[pl.ds(wid * 2 * _L, 2 * _L)],
                        buf_v.at[pl.ds(_LANES, 2 * _L)])

        lane15 = jnp.full((_LANES,), 15, jnp.int32)

        def body(i, carry):
            lz_v, n_v = carry
            base = i * _LANES
            idx = lax.iota(jnp.int32, _LANES) + base
            m_cur = buf_v[pl.ds(base + _M0, _LANES)]
            m_prev = buf_v[pl.ds(base + _M0 - 1, _LANES)]
            t_prev = buf_v[pl.ds(base + _T0 - 1, _LANES)]
            a = (m_cur == 1) & (m_prev == 1) & (t_prev <= _EXT_MAX_ID)
            v = jnp.where(a, -1, idx)
            y = jnp.maximum(plsc.cummax(v), lz_v)
            n = 1 - ((idx - y) & 1)
            c = plsc.cumsum(n) + n_v
            out_v[pl.ds(base, _LANES)] = c - 1
            return (y.at[lane15].get(mode="promise_in_bounds"),
                    c.at[lane15].get(mode="promise_in_bounds"))

        lax.fori_loop(0, _CHUNKS, body,
                      (jnp.full((_LANES,), -1, jnp.int32),
                       jnp.zeros((_LANES,), jnp.int32)), unroll=1)
        pltpu.sync_copy(out_v, out_hbm.at[pl.ds(wid * _L, _L)])


_sc_chunker = functools.partial(
    pl.kernel,
    out_type=jax.ShapeDtypeStruct((_B * _L,), jnp.int32),
    mesh=plsc.VectorSubcoreMesh(core_axis_name="c", subcore_axis_name="s",
                                num_cores=1, num_subcores=16),
    compiler_params=pltpu.CompilerParams(needs_layout_passes=False,
                                         skip_device_barrier=True,
                                         disable_bounds_checks=True,
                                         disable_semaphore_checks=True),
    scratch_types=[
        pltpu.VMEM((2 * _L + 2 * _LANES,), jnp.int32),
        pltpu.VMEM((_L,), jnp.int32),
    ],
)(_sc_body)


def kernel(inp, regular_tokens_mask, token_ids):
    del inp
    mt = jnp.stack([regular_tokens_mask, token_ids],
                   axis=1).reshape(2 * _B * _L)
    return _sc_chunker(mt).reshape(_B, _L)


# speedup vs baseline: 1.0120x; 1.0000x over previous
import functools

import jax
import jax.numpy as jnp
from jax import lax
from jax.experimental import pallas as pl
from jax.experimental.pallas import tpu as pltpu
from jax.experimental.pallas import tpu_sc as plsc

_B, _L = 8, 2048
_LANES = 16
_CHUNKS = _L // _LANES
_EXT_MAX_ID = 20030

_M0 = _LANES
_T0 = _LANES + _L


def _sc_body(mt_hbm, out_hbm, buf_v, out_v, sem):
    wid = lax.axis_index("s")

    @pl.when(wid < _B)
    def _():
        buf_v[pl.ds(0, _LANES)] = jnp.zeros((_LANES,), jnp.int32)
        pltpu.sync_copy(mt_hbm.at[pl.ds(wid * 2 * _L, 2 * _L)],
                        buf_v.at[pl.ds(_LANES, 2 * _L)])

        lane15 = jnp.full((_LANES,), 15, jnp.int32)

        def body(i, carry):
            lz_v, n_v = carry
            base = i * _LANES
            idx = lax.iota(jnp.int32, _LANES) + base
            m_cur = buf_v[pl.ds(base + _M0, _LANES)]
            m_prev = buf_v[pl.ds(base + _M0 - 1, _LANES)]
            t_prev = buf_v[pl.ds(base + _T0 - 1, _LANES)]
            a = (m_cur == 1) & (m_prev == 1) & (t_prev <= _EXT_MAX_ID)
            v = jnp.where(a, -1, idx)
            y = jnp.maximum(plsc.cummax(v), lz_v)
            n = 1 - ((idx - y) & 1)
            c = plsc.cumsum(n) + n_v
            out_v[pl.ds(base, _LANES)] = c - 1
            return (y.at[lane15].get(mode="promise_in_bounds"),
                    c.at[lane15].get(mode="promise_in_bounds"))

        blk = _CHUNKS // 4

        def outer(b, carry):
            def inner(i, carry):
                return body(b * blk + i, carry)
            carry = lax.fori_loop(0, blk, inner, carry, unroll=1)
            off = b * blk * _LANES
            pltpu.async_copy(out_v.at[pl.ds(off, blk * _LANES)],
                             out_hbm.at[pl.ds(wid * _L + off, blk * _LANES)],
                             sem)
            return carry

        lax.fori_loop(0, 4, outer,
                      (jnp.full((_LANES,), -1, jnp.int32),
                       jnp.zeros((_LANES,), jnp.int32)), unroll=1)
        pltpu.make_async_copy(out_hbm.at[pl.ds(wid * _L, _L)], out_v,
                              sem).wait()


_sc_chunker = functools.partial(
    pl.kernel,
    out_type=jax.ShapeDtypeStruct((_B * _L,), jnp.int32),
    mesh=plsc.VectorSubcoreMesh(core_axis_name="c", subcore_axis_name="s",
                                num_cores=1, num_subcores=16),
    compiler_params=pltpu.CompilerParams(needs_layout_passes=False,
                                         skip_device_barrier=True,
                                         disable_bounds_checks=True,
                                         disable_semaphore_checks=True),
    scratch_types=[
        pltpu.VMEM((2 * _L + 2 * _LANES,), jnp.int32),
        pltpu.VMEM((_L,), jnp.int32),
        pltpu.SemaphoreType.DMA,
    ],
)(_sc_body)


def kernel(inp, regular_tokens_mask, token_ids):
    del inp
    mt = jnp.stack([regular_tokens_mask, token_ids],
                   axis=1).reshape(2 * _B * _L)
    return _sc_chunker(mt).reshape(_B, _L)
